# baseline (device time: 196201 ns/iter reference)
import jax
import jax.numpy as jnp
from jax import lax
from jax.experimental import pallas as pl
from jax.experimental.pallas import tpu as pltpu

M, N = 16384, 1024
PH = 2752
PCH = PH // 4
DQ = 2720
DCH = DQ // 4
DBASE = 2 * PH
assert DBASE + 4 * DQ == M


def kernel(x):
    def body(x_ref, out_ref,
             svd, svp, yvd, yvp, xrd, zrd, dfx, dfz, pxr, pzr,
             xfd, xfp,
             f32d_sems, f32p_sems,
             ysend, yrecv, xsend, xrecv, zsend, zrecv,
             pxsend, pxrecv, pzsend, pzrecv,
             fxsend, fxrecv, fzsend, fzrecv,
             o_dl, o_pl, o_x, o_z, o_px, o_pz, o_dx, o_dz):
        my_x = lax.axis_index("x")
        my_y = lax.axis_index("y")
        my_z = lax.axis_index("z")
        py_id = (my_x, 1 - my_y, my_z)
        px_id = (1 - my_x, my_y, my_z)
        pz_id = (my_x, my_y, 1 - my_z)

        q = 2 * my_x + my_z
        qx = 2 * (1 - my_x) + my_z
        qz = 2 * my_x + (1 - my_z)
        qd = 2 * (1 - my_x) + (1 - my_z)
        pr = (my_x + my_z) % 2
        myd0 = DBASE + q * DQ
        myp0 = pr * PH

        barrier_sem = pltpu.get_barrier_semaphore()
        for nbr in (py_id, px_id, pz_id):
            pl.semaphore_signal(
                barrier_sem, inc=1, device_id=nbr,
                device_id_type=pl.DeviceIdType.MESH,
            )

        def dds(c):
            return (pl.ds(c * DCH, DCH), slice(None))

        def pds(c):
            return (pl.ds(c * PCH, PCH), slice(None))

        yd_rdmas, yp_rdmas = [], []
        pltpu.make_async_copy(
            x_ref.at[pl.ds(myd0, DCH), :], xfd.at[0], f32d_sems.at[0]
        ).start()
        pl.semaphore_wait(barrier_sem, 3)
        for c in range(4):
            pltpu.make_async_copy(
                x_ref.at[pl.ds(myd0 + c * DCH, DCH), :],
                xfd.at[c % 2], f32d_sems.at[c % 2],
            ).wait()
            if c + 1 < 4:
                pltpu.make_async_copy(
                    x_ref.at[pl.ds(myd0 + (c + 1) * DCH, DCH), :],
                    xfd.at[(c + 1) % 2], f32d_sems.at[(c + 1) % 2],
                ).start()
            svd[dds(c)] = xfd[c % 2].astype(jnp.bfloat16)
            r = pltpu.make_async_remote_copy(
                src_ref=svd.at[dds(c)], dst_ref=yvd.at[dds(c)],
                send_sem=ysend.at[c], recv_sem=yrecv.at[c],
                device_id=py_id, device_id_type=pl.DeviceIdType.MESH,
            )
            r.start()
            yd_rdmas.append(r)
        pltpu.make_async_copy(
            x_ref.at[pl.ds(myp0, PCH), :], xfp.at[0], f32p_sems.at[0]
        ).start()
        for c in range(4):
            pltpu.make_async_copy(
                x_ref.at[pl.ds(myp0 + c * PCH, PCH), :],
                xfp.at[c % 2], f32p_sems.at[c % 2],
            ).wait()
            if c + 1 < 4:
                pltpu.make_async_copy(
                    x_ref.at[pl.ds(myp0 + (c + 1) * PCH, PCH), :],
                    xfp.at[(c + 1) % 2], f32p_sems.at[(c + 1) % 2],
                ).start()
            svp[pds(c)] = xfp[c % 2].astype(jnp.bfloat16)
            r = pltpu.make_async_remote_copy(
                src_ref=svp.at[pds(c)], dst_ref=yvp.at[pds(c)],
                send_sem=ysend.at[4 + c], recv_sem=yrecv.at[4 + c],
                device_id=py_id, device_id_type=pl.DeviceIdType.MESH,
            )
            r.start()
            yp_rdmas.append(r)

        x_own, z_own = [], []
        for c in range(4):
            yd_rdmas[c].wait_recv()
            yvd[dds(c)] = yvd[dds(c)] + svd[dds(c)]
            rx = pltpu.make_async_remote_copy(
                src_ref=yvd.at[dds(c)], dst_ref=xrd.at[dds(c)],
                send_sem=xsend.at[c], recv_sem=xrecv.at[c],
                device_id=px_id, device_id_type=pl.DeviceIdType.MESH,
            )
            rx.start()
            x_own.append(rx)
            rz = pltpu.make_async_remote_copy(
                src_ref=yvd.at[dds(c)], dst_ref=zrd.at[dds(c)],
                send_sem=zsend.at[c], recv_sem=zrecv.at[c],
                device_id=pz_id, device_id_type=pl.DeviceIdType.MESH,
            )
            rz.start()
            z_own.append(rz)
            pltpu.make_async_copy(
                yvd.at[dds(c)],
                out_ref.at[pl.ds(myd0 + c * DCH, DCH), :],
                o_dl.at[c],
            ).start()

        p_sends = []
        for c in range(4):
            yp_rdmas[c].wait_recv()
            yvp[pds(c)] = yvp[pds(c)] + svp[pds(c)]
            if c < 2:
                r = pltpu.make_async_remote_copy(
                    src_ref=yvp.at[pds(c)], dst_ref=pxr.at[pds(c)],
                    send_sem=pxsend.at[c], recv_sem=pxrecv.at[c],
                    device_id=px_id, device_id_type=pl.DeviceIdType.MESH,
                )
            else:
                r = pltpu.make_async_remote_copy(
                    src_ref=yvp.at[pds(c)], dst_ref=pzr.at[pds(c - 2)],
                    send_sem=pzsend.at[c - 2], recv_sem=pzrecv.at[c - 2],
                    device_id=pz_id, device_id_type=pl.DeviceIdType.MESH,
                )
            r.start()
            p_sends.append(r)
            pltpu.make_async_copy(
                yvp.at[pds(c)],
                out_ref.at[pl.ds(myp0 + c * PCH, PCH), :],
                o_pl.at[c],
            ).start()

        fx_list, fz_list = [], []
        for c in range(4):
            z_own[c].wait_recv()
            pltpu.make_async_copy(
                zrd.at[dds(c)],
                out_ref.at[pl.ds(DBASE + qz * DQ + c * DCH, DCH), :],
                o_z.at[c],
            ).start()
            if c < 2:
                f = pltpu.make_async_remote_copy(
                    src_ref=zrd.at[dds(c)], dst_ref=dfx.at[dds(c)],
                    send_sem=fxsend.at[c], recv_sem=fxrecv.at[c],
                    device_id=px_id, device_id_type=pl.DeviceIdType.MESH,
                )
                f.start()
                fx_list.append(f)
            x_own[c].wait_recv()
            pltpu.make_async_copy(
                xrd.at[dds(c)],
                out_ref.at[pl.ds(DBASE + qx * DQ + c * DCH, DCH), :],
                o_x.at[c],
            ).start()
            if c >= 2:
                j = c - 2
                f = pltpu.make_async_remote_copy(
                    src_ref=xrd.at[dds(c)], dst_ref=dfz.at[dds(j)],
                    send_sem=fzsend.at[j], recv_sem=fzrecv.at[j],
                    device_id=pz_id, device_id_type=pl.DeviceIdType.MESH,
                )
                f.start()
                fz_list.append(f)

        otherp0 = (1 - pr) * PH
        for j in range(2):
            p_sends[j].wait_recv()
            pltpu.make_async_copy(
                pxr.at[pds(j)],
                out_ref.at[pl.ds(otherp0 + j * PCH, PCH), :],
                o_px.at[j],
            ).start()
            p_sends[2 + j].wait_recv()
            pltpu.make_async_copy(
                pzr.at[pds(j)],
                out_ref.at[pl.ds(otherp0 + (2 + j) * PCH, PCH), :],
                o_pz.at[j],
            ).start()

        for j in range(2):
            fx_list[j].wait_recv()
            pltpu.make_async_copy(
                dfx.at[dds(j)],
                out_ref.at[pl.ds(DBASE + qd * DQ + j * DCH, DCH), :],
                o_dx.at[j],
            ).start()
            fz_list[j].wait_recv()
            pltpu.make_async_copy(
                dfz.at[dds(j)],
                out_ref.at[pl.ds(DBASE + qd * DQ + (2 + j) * DCH, DCH), :],
                o_dz.at[j],
            ).start()

        for c in range(4):
            yd_rdmas[c].wait_send()
            yp_rdmas[c].wait_send()
            x_own[c].wait_send()
            z_own[c].wait_send()
            p_sends[c].wait_send()
            pltpu.make_async_copy(
                yvd.at[dds(c)],
                out_ref.at[pl.ds(myd0 + c * DCH, DCH), :],
                o_dl.at[c],
            ).wait()
            pltpu.make_async_copy(
                yvp.at[pds(c)],
                out_ref.at[pl.ds(myp0 + c * PCH, PCH), :],
                o_pl.at[c],
            ).wait()
            pltpu.make_async_copy(
                zrd.at[dds(c)],
                out_ref.at[pl.ds(DBASE + qz * DQ + c * DCH, DCH), :],
                o_z.at[c],
            ).wait()
            pltpu.make_async_copy(
                xrd.at[dds(c)],
                out_ref.at[pl.ds(DBASE + qx * DQ + c * DCH, DCH), :],
                o_x.at[c],
            ).wait()
        for j in range(2):
            fx_list[j].wait_send()
            fz_list[j].wait_send()
            pltpu.make_async_copy(
                pxr.at[pds(j)],
                out_ref.at[pl.ds(otherp0 + j * PCH, PCH), :],
                o_px.at[j],
            ).wait()
            pltpu.make_async_copy(
                pzr.at[pds(j)],
                out_ref.at[pl.ds(otherp0 + (2 + j) * PCH, PCH), :],
                o_pz.at[j],
            ).wait()
            pltpu.make_async_copy(
                dfx.at[dds(j)],
                out_ref.at[pl.ds(DBASE + qd * DQ + j * DCH, DCH), :],
                o_dx.at[j],
            ).wait()
            pltpu.make_async_copy(
                dfz.at[dds(j)],
                out_ref.at[pl.ds(DBASE + qd * DQ + (2 + j) * DCH, DCH), :],
                o_dz.at[j],
            ).wait()

    return pl.pallas_call(
        body,
        out_shape=jax.ShapeDtypeStruct((M, N), jnp.bfloat16),
        in_specs=[pl.BlockSpec(memory_space=pltpu.HBM)],
        out_specs=pl.BlockSpec(memory_space=pltpu.HBM),
        scratch_shapes=[
            pltpu.VMEM((DQ, N), jnp.bfloat16),
            pltpu.VMEM((PH, N), jnp.bfloat16),
            pltpu.VMEM((DQ, N), jnp.bfloat16),
            pltpu.VMEM((PH, N), jnp.bfloat16),
            pltpu.VMEM((DQ, N), jnp.bfloat16),
            pltpu.VMEM((DQ, N), jnp.bfloat16),
            pltpu.VMEM((DQ // 2, N), jnp.bfloat16),
            pltpu.VMEM((DQ // 2, N), jnp.bfloat16),
            pltpu.VMEM((PH // 2, N), jnp.bfloat16),
            pltpu.VMEM((PH // 2, N), jnp.bfloat16),
            pltpu.VMEM((2, DCH, N), jnp.float32),
            pltpu.VMEM((2, PCH, N), jnp.float32),
            pltpu.SemaphoreType.DMA((2,)),
            pltpu.SemaphoreType.DMA((2,)),
            pltpu.SemaphoreType.DMA((8,)),
            pltpu.SemaphoreType.DMA((8,)),
            pltpu.SemaphoreType.DMA((4,)),
            pltpu.SemaphoreType.DMA((4,)),
            pltpu.SemaphoreType.DMA((4,)),
            pltpu.SemaphoreType.DMA((4,)),
            pltpu.SemaphoreType.DMA((2,)),
            pltpu.SemaphoreType.DMA((2,)),
            pltpu.SemaphoreType.DMA((2,)),
            pltpu.SemaphoreType.DMA((2,)),
            pltpu.SemaphoreType.DMA((2,)),
            pltpu.SemaphoreType.DMA((2,)),
            pltpu.SemaphoreType.DMA((2,)),
            pltpu.SemaphoreType.DMA((2,)),
            pltpu.SemaphoreType.DMA((4,)),
            pltpu.SemaphoreType.DMA((4,)),
            pltpu.SemaphoreType.DMA((4,)),
            pltpu.SemaphoreType.DMA((4,)),
            pltpu.SemaphoreType.DMA((2,)),
            pltpu.SemaphoreType.DMA((2,)),
            pltpu.SemaphoreType.DMA((2,)),
            pltpu.SemaphoreType.DMA((2,)),
        ],
        compiler_params=pltpu.CompilerParams(
            collective_id=0,
            vmem_limit_bytes=62 * 1024 * 1024,
        ),
    )(x)


# device time: 189762 ns/iter; 1.0339x vs baseline; 1.0339x over previous
import jax
import jax.numpy as jnp
from jax import lax
from jax.experimental import pallas as pl
from jax.experimental.pallas import tpu as pltpu

M, N = 16384, 1024
QR = M // 4
CH = 1024
Kq = QR // CH
KH = Kq // 2


def kernel(x):
    def body(x_ref, out_ref, sv, yv, xrv, zrv, dfx, dfz, xf,
             f32_sems, ysend, yrecv, xsend, xrecv, zsend, zrecv,
             fxsend, fxrecv, fzsend, fzrecv,
             o_own, o_x, o_z, o_dx, o_dz):
        my_x = lax.axis_index("x")
        my_y = lax.axis_index("y")
        my_z = lax.axis_index("z")
        py_id = (my_x, 1 - my_y, my_z)
        px_id = (1 - my_x, my_y, my_z)
        pz_id = (my_x, my_y, 1 - my_z)

        q = 2 * my_x + my_z
        qx = 2 * (1 - my_x) + my_z
        qz = 2 * my_x + (1 - my_z)
        qd = 2 * (1 - my_x) + (1 - my_z)
        myq0 = q * QR

        barrier_sem = pltpu.get_barrier_semaphore()
        for nbr in (py_id, px_id, pz_id):
            pl.semaphore_signal(
                barrier_sem, inc=1, device_id=nbr,
                device_id_type=pl.DeviceIdType.MESH,
            )

        def ds(c):
            return (pl.ds(c * CH, CH), slice(None))

        y_rdmas = []
        pltpu.make_async_copy(
            x_ref.at[pl.ds(myq0, CH), :], xf.at[0], f32_sems.at[0]
        ).start()
        pl.semaphore_wait(barrier_sem, 3)
        for c in range(Kq):
            pltpu.make_async_copy(
                x_ref.at[pl.ds(myq0 + c * CH, CH), :],
                xf.at[c % 2], f32_sems.at[c % 2],
            ).wait()
            if c + 1 < Kq:
                pltpu.make_async_copy(
                    x_ref.at[pl.ds(myq0 + (c + 1) * CH, CH), :],
                    xf.at[(c + 1) % 2], f32_sems.at[(c + 1) % 2],
                ).start()
            sv[ds(c)] = xf[c % 2].astype(jnp.bfloat16)
            r = pltpu.make_async_remote_copy(
                src_ref=sv.at[ds(c)], dst_ref=yv.at[ds(c)],
                send_sem=ysend.at[c], recv_sem=yrecv.at[c],
                device_id=py_id, device_id_type=pl.DeviceIdType.MESH,
            )
            r.start()
            y_rdmas.append(r)

        x_own, z_own = [], []
        for c in range(Kq):
            y_rdmas[c].wait_recv()
            yv[ds(c)] = yv[ds(c)] + sv[ds(c)]
            rx = pltpu.make_async_remote_copy(
                src_ref=yv.at[ds(c)], dst_ref=xrv.at[ds(c)],
                send_sem=xsend.at[c], recv_sem=xrecv.at[c],
                device_id=px_id, device_id_type=pl.DeviceIdType.MESH,
            )
            rx.start()
            x_own.append(rx)
            rz = pltpu.make_async_remote_copy(
                src_ref=yv.at[ds(c)], dst_ref=zrv.at[ds(c)],
                send_sem=zsend.at[c], recv_sem=zrecv.at[c],
                device_id=pz_id, device_id_type=pl.DeviceIdType.MESH,
            )
            rz.start()
            z_own.append(rz)
            pltpu.make_async_copy(
                yv.at[ds(c)],
                out_ref.at[pl.ds(myq0 + c * CH, CH), :],
                o_own.at[c],
            ).start()

        fx_list, fz_list = [], []
        for c in range(Kq):
            z_own[c].wait_recv()
            pltpu.make_async_copy(
                zrv.at[ds(c)],
                out_ref.at[pl.ds(qz * QR + c * CH, CH), :],
                o_z.at[c],
            ).start()
            if c < KH:
                f = pltpu.make_async_remote_copy(
                    src_ref=zrv.at[ds(c)], dst_ref=dfx.at[ds(c)],
                    send_sem=fxsend.at[c], recv_sem=fxrecv.at[c],
                    device_id=px_id, device_id_type=pl.DeviceIdType.MESH,
                )
                f.start()
                fx_list.append(f)
            x_own[c].wait_recv()
            pltpu.make_async_copy(
                xrv.at[ds(c)],
                out_ref.at[pl.ds(qx * QR + c * CH, CH), :],
                o_x.at[c],
            ).start()
            if c >= KH:
                j = c - KH
                f = pltpu.make_async_remote_copy(
                    src_ref=xrv.at[ds(c)], dst_ref=dfz.at[ds(j)],
                    send_sem=fzsend.at[j], recv_sem=fzrecv.at[j],
                    device_id=pz_id, device_id_type=pl.DeviceIdType.MESH,
                )
                f.start()
                fz_list.append(f)

        for c in range(KH):
            fx_list[c].wait_recv()
            pltpu.make_async_copy(
                dfx.at[ds(c)],
                out_ref.at[pl.ds(qd * QR + c * CH, CH), :],
                o_dx.at[c],
            ).start()
            fz_list[c].wait_recv()
            pltpu.make_async_copy(
                dfz.at[ds(c)],
                out_ref.at[pl.ds(qd * QR + (KH + c) * CH, CH), :],
                o_dz.at[c],
            ).start()

        for c in range(Kq):
            y_rdmas[c].wait_send()
            x_own[c].wait_send()
            z_own[c].wait_send()
            pltpu.make_async_copy(
                yv.at[ds(c)],
                out_ref.at[pl.ds(myq0 + c * CH, CH), :],
                o_own.at[c],
            ).wait()
            pltpu.make_async_copy(
                zrv.at[ds(c)],
                out_ref.at[pl.ds(qz * QR + c * CH, CH), :],
                o_z.at[c],
            ).wait()
            pltpu.make_async_copy(
                xrv.at[ds(c)],
                out_ref.at[pl.ds(qx * QR + c * CH, CH), :],
                o_x.at[c],
            ).wait()
        for c in range(KH):
            fx_list[c].wait_send()
            fz_list[c].wait_send()
            pltpu.make_async_copy(
                dfx.at[ds(c)],
                out_ref.at[pl.ds(qd * QR + c * CH, CH), :],
                o_dx.at[c],
            ).wait()
            pltpu.make_async_copy(
                dfz.at[ds(c)],
                out_ref.at[pl.ds(qd * QR + (KH + c) * CH, CH), :],
                o_dz.at[c],
            ).wait()

    return pl.pallas_call(
        body,
        out_shape=jax.ShapeDtypeStruct((M, N), jnp.bfloat16),
        in_specs=[pl.BlockSpec(memory_space=pltpu.HBM)],
        out_specs=pl.BlockSpec(memory_space=pltpu.HBM),
        scratch_shapes=[
            pltpu.VMEM((QR, N), jnp.bfloat16),
            pltpu.VMEM((QR, N), jnp.bfloat16),
            pltpu.VMEM((QR, N), jnp.bfloat16),
            pltpu.VMEM((QR, N), jnp.bfloat16),
            pltpu.VMEM((QR // 2, N), jnp.bfloat16),
            pltpu.VMEM((QR // 2, N), jnp.bfloat16),
            pltpu.VMEM((2, CH, N), jnp.float32),
            pltpu.SemaphoreType.DMA((2,)),
            pltpu.SemaphoreType.DMA((Kq,)),
            pltpu.SemaphoreType.DMA((Kq,)),
            pltpu.SemaphoreType.DMA((Kq,)),
            pltpu.SemaphoreType.DMA((Kq,)),
            pltpu.SemaphoreType.DMA((Kq,)),
            pltpu.SemaphoreType.DMA((Kq,)),
            pltpu.SemaphoreType.DMA((KH,)),
            pltpu.SemaphoreType.DMA((KH,)),
            pltpu.SemaphoreType.DMA((KH,)),
            pltpu.SemaphoreType.DMA((KH,)),
            pltpu.SemaphoreType.DMA((Kq,)),
            pltpu.SemaphoreType.DMA((Kq,)),
            pltpu.SemaphoreType.DMA((Kq,)),
            pltpu.SemaphoreType.DMA((KH,)),
            pltpu.SemaphoreType.DMA((KH,)),
        ],
        compiler_params=pltpu.CompilerParams(
            collective_id=0,
            vmem_limit_bytes=62 * 1024 * 1024,
        ),
    )(x)


# device time: 183359 ns/iter; 1.0700x vs baseline; 1.0349x over previous
import jax
import jax.numpy as jnp
from jax import lax
from jax.experimental import pallas as pl
from jax.experimental.pallas import tpu as pltpu

M, N = 16384, 1024
QR = M // 4
CH = 256
Kq = QR // CH
KH = Kq // 2


def kernel(x):
    def body(x_ref, out_ref, sv, yv, xrv, zrv, dfx, dfz, xf,
             f32_sems, ysend, yrecv, xsend, xrecv, zsend, zrecv,
             fxsend, fxrecv, fzsend, fzrecv,
             o_own, o_x, o_z, o_dx, o_dz):
        my_x = lax.axis_index("x")
        my_y = lax.axis_index("y")
        my_z = lax.axis_index("z")
        py_id = (my_x, 1 - my_y, my_z)
        px_id = (1 - my_x, my_y, my_z)
        pz_id = (my_x, my_y, 1 - my_z)

        q = 2 * my_x + my_z
        qx = 2 * (1 - my_x) + my_z
        qz = 2 * my_x + (1 - my_z)
        qd = 2 * (1 - my_x) + (1 - my_z)
        myq0 = q * QR

        barrier_sem = pltpu.get_barrier_semaphore()
        for nbr in (py_id, px_id, pz_id):
            pl.semaphore_signal(
                barrier_sem, inc=1, device_id=nbr,
                device_id_type=pl.DeviceIdType.MESH,
            )

        def ds(c):
            return (pl.ds(c * CH, CH), slice(None))

        y_rdmas = []
        pltpu.make_async_copy(
            x_ref.at[pl.ds(myq0, CH), :], xf.at[0], f32_sems.at[0]
        ).start()
        pl.semaphore_wait(barrier_sem, 3)
        for c in range(Kq):
            pltpu.make_async_copy(
                x_ref.at[pl.ds(myq0 + c * CH, CH), :],
                xf.at[c % 2], f32_sems.at[c % 2],
            ).wait()
            if c + 1 < Kq:
                pltpu.make_async_copy(
                    x_ref.at[pl.ds(myq0 + (c + 1) * CH, CH), :],
                    xf.at[(c + 1) % 2], f32_sems.at[(c + 1) % 2],
                ).start()
            sv[ds(c)] = xf[c % 2].astype(jnp.bfloat16)
            r = pltpu.make_async_remote_copy(
                src_ref=sv.at[ds(c)], dst_ref=yv.at[ds(c)],
                send_sem=ysend.at[c], recv_sem=yrecv.at[c],
                device_id=py_id, device_id_type=pl.DeviceIdType.MESH,
            )
            r.start()
            y_rdmas.append(r)

        x_own, z_own = [], []
        for c in range(Kq):
            y_rdmas[c].wait_recv()
            yv[ds(c)] = yv[ds(c)] + sv[ds(c)]
            rx = pltpu.make_async_remote_copy(
                src_ref=yv.at[ds(c)], dst_ref=xrv.at[ds(c)],
                send_sem=xsend.at[c], recv_sem=xrecv.at[c],
                device_id=px_id, device_id_type=pl.DeviceIdType.MESH,
            )
            rx.start()
            x_own.append(rx)
            rz = pltpu.make_async_remote_copy(
                src_ref=yv.at[ds(c)], dst_ref=zrv.at[ds(c)],
                send_sem=zsend.at[c], recv_sem=zrecv.at[c],
                device_id=pz_id, device_id_type=pl.DeviceIdType.MESH,
            )
            rz.start()
            z_own.append(rz)
            pltpu.make_async_copy(
                yv.at[ds(c)],
                out_ref.at[pl.ds(myq0 + c * CH, CH), :],
                o_own.at[c],
            ).start()

        fx_list, fz_list = [], []
        for c in range(Kq):
            z_own[c].wait_recv()
            pltpu.make_async_copy(
                zrv.at[ds(c)],
                out_ref.at[pl.ds(qz * QR + c * CH, CH), :],
                o_z.at[c],
            ).start()
            if c < KH:
                f = pltpu.make_async_remote_copy(
                    src_ref=zrv.at[ds(c)], dst_ref=dfx.at[ds(c)],
                    send_sem=fxsend.at[c], recv_sem=fxrecv.at[c],
                    device_id=px_id, device_id_type=pl.DeviceIdType.MESH,
                )
                f.start()
                fx_list.append(f)
            x_own[c].wait_recv()
            pltpu.make_async_copy(
                xrv.at[ds(c)],
                out_ref.at[pl.ds(qx * QR + c * CH, CH), :],
                o_x.at[c],
            ).start()
            if c >= KH:
                j = c - KH
                f = pltpu.make_async_remote_copy(
                    src_ref=xrv.at[ds(c)], dst_ref=dfz.at[ds(j)],
                    send_sem=fzsend.at[j], recv_sem=fzrecv.at[j],
                    device_id=pz_id, device_id_type=pl.DeviceIdType.MESH,
                )
                f.start()
                fz_list.append(f)

        for c in range(KH):
            fx_list[c].wait_recv()
            pltpu.make_async_copy(
                dfx.at[ds(c)],
                out_ref.at[pl.ds(qd * QR + c * CH, CH), :],
                o_dx.at[c],
            ).start()
            fz_list[c].wait_recv()
            pltpu.make_async_copy(
                dfz.at[ds(c)],
                out_ref.at[pl.ds(qd * QR + (KH + c) * CH, CH), :],
                o_dz.at[c],
            ).start()

        for c in range(Kq):
            y_rdmas[c].wait_send()
            x_own[c].wait_send()
            z_own[c].wait_send()
            pltpu.make_async_copy(
                yv.at[ds(c)],
                out_ref.at[pl.ds(myq0 + c * CH, CH), :],
                o_own.at[c],
            ).wait()
            pltpu.make_async_copy(
                zrv.at[ds(c)],
                out_ref.at[pl.ds(qz * QR + c * CH, CH), :],
                o_z.at[c],
            ).wait()
            pltpu.make_async_copy(
                xrv.at[ds(c)],
                out_ref.at[pl.ds(qx * QR + c * CH, CH), :],
                o_x.at[c],
            ).wait()
        for c in range(KH):
            fx_list[c].wait_send()
            fz_list[c].wait_send()
            pltpu.make_async_copy(
                dfx.at[ds(c)],
                out_ref.at[pl.ds(qd * QR + c * CH, CH), :],
                o_dx.at[c],
            ).wait()
            pltpu.make_async_copy(
                dfz.at[ds(c)],
                out_ref.at[pl.ds(qd * QR + (KH + c) * CH, CH), :],
                o_dz.at[c],
            ).wait()

    return pl.pallas_call(
        body,
        out_shape=jax.ShapeDtypeStruct((M, N), jnp.bfloat16),
        in_specs=[pl.BlockSpec(memory_space=pltpu.HBM)],
        out_specs=pl.BlockSpec(memory_space=pltpu.HBM),
        scratch_shapes=[
            pltpu.VMEM((QR, N), jnp.bfloat16),
            pltpu.VMEM((QR, N), jnp.bfloat16),
            pltpu.VMEM((QR, N), jnp.bfloat16),
            pltpu.VMEM((QR, N), jnp.bfloat16),
            pltpu.VMEM((QR // 2, N), jnp.bfloat16),
            pltpu.VMEM((QR // 2, N), jnp.bfloat16),
            pltpu.VMEM((2, CH, N), jnp.float32),
            pltpu.SemaphoreType.DMA((2,)),
            pltpu.SemaphoreType.DMA((Kq,)),
            pltpu.SemaphoreType.DMA((Kq,)),
            pltpu.SemaphoreType.DMA((Kq,)),
            pltpu.SemaphoreType.DMA((Kq,)),
            pltpu.SemaphoreType.DMA((Kq,)),
            pltpu.SemaphoreType.DMA((Kq,)),
            pltpu.SemaphoreType.DMA((KH,)),
            pltpu.SemaphoreType.DMA((KH,)),
            pltpu.SemaphoreType.DMA((KH,)),
            pltpu.SemaphoreType.DMA((KH,)),
            pltpu.SemaphoreType.DMA((Kq,)),
            pltpu.SemaphoreType.DMA((Kq,)),
            pltpu.SemaphoreType.DMA((Kq,)),
            pltpu.SemaphoreType.DMA((KH,)),
            pltpu.SemaphoreType.DMA((KH,)),
        ],
        compiler_params=pltpu.CompilerParams(
            collective_id=0,
            vmem_limit_bytes=62 * 1024 * 1024,
        ),
    )(x)


# device time: 177095 ns/iter; 1.1079x vs baseline; 1.0354x over previous
import jax
import jax.numpy as jnp
from jax import lax
from jax.experimental import pallas as pl
from jax.experimental.pallas import tpu as pltpu

M, N = 16384, 1024
QR = M // 4
SIZES = [256, 256, 512, 512, 512, 512, 512, 512, 256, 256]
STARTS = [sum(SIZES[:i]) for i in range(len(SIZES))]
assert sum(SIZES) == QR
Kq = len(SIZES)
KH = next(i for i, s in enumerate(STARTS) if s >= QR // 2)
CHMAX = max(SIZES)


def kernel(x):
    def body(x_ref, out_ref, sv, yv, xrv, zrv, dfx, dfz, xf,
             f32_sems, ysend, yrecv, xsend, xrecv, zsend, zrecv,
             fxsend, fxrecv, fzsend, fzrecv,
             o_own, o_x, o_z, o_dx, o_dz):
        my_x = lax.axis_index("x")
        my_y = lax.axis_index("y")
        my_z = lax.axis_index("z")
        py_id = (my_x, 1 - my_y, my_z)
        px_id = (1 - my_x, my_y, my_z)
        pz_id = (my_x, my_y, 1 - my_z)

        q = 2 * my_x + my_z
        qx = 2 * (1 - my_x) + my_z
        qz = 2 * my_x + (1 - my_z)
        qd = 2 * (1 - my_x) + (1 - my_z)
        myq0 = q * QR

        barrier_sem = pltpu.get_barrier_semaphore()
        for nbr in (py_id, px_id, pz_id):
            pl.semaphore_signal(
                barrier_sem, inc=1, device_id=nbr,
                device_id_type=pl.DeviceIdType.MESH,
            )

        def ds(c):
            return (pl.ds(STARTS[c], SIZES[c]), slice(None))

        def xfs(c):
            return (pl.ds((c % 2) * CHMAX, SIZES[c]), slice(None))

        y_rdmas = []
        pltpu.make_async_copy(
            x_ref.at[pl.ds(myq0, SIZES[0]), :], xf.at[xfs(0)], f32_sems.at[0]
        ).start()
        pl.semaphore_wait(barrier_sem, 3)
        for c in range(Kq):
            pltpu.make_async_copy(
                x_ref.at[pl.ds(myq0 + STARTS[c], SIZES[c]), :],
                xf.at[xfs(c)], f32_sems.at[c % 2],
            ).wait()
            if c + 1 < Kq:
                pltpu.make_async_copy(
                    x_ref.at[pl.ds(myq0 + STARTS[c + 1], SIZES[c + 1]), :],
                    xf.at[xfs(c + 1)], f32_sems.at[(c + 1) % 2],
                ).start()
            sv[ds(c)] = xf[xfs(c)].astype(jnp.bfloat16)
            r = pltpu.make_async_remote_copy(
                src_ref=sv.at[ds(c)], dst_ref=yv.at[ds(c)],
                send_sem=ysend.at[c], recv_sem=yrecv.at[c],
                device_id=py_id, device_id_type=pl.DeviceIdType.MESH,
            )
            r.start()
            y_rdmas.append(r)

        x_own, z_own = [], []
        for c in range(Kq):
            y_rdmas[c].wait_recv()
            yv[ds(c)] = yv[ds(c)] + sv[ds(c)]
            rx = pltpu.make_async_remote_copy(
                src_ref=yv.at[ds(c)], dst_ref=xrv.at[ds(c)],
                send_sem=xsend.at[c], recv_sem=xrecv.at[c],
                device_id=px_id, device_id_type=pl.DeviceIdType.MESH,
            )
            rx.start()
            x_own.append(rx)
            rz = pltpu.make_async_remote_copy(
                src_ref=yv.at[ds(c)], dst_ref=zrv.at[ds(c)],
                send_sem=zsend.at[c], recv_sem=zrecv.at[c],
                device_id=pz_id, device_id_type=pl.DeviceIdType.MESH,
            )
            rz.start()
            z_own.append(rz)
            pltpu.make_async_copy(
                yv.at[ds(c)],
                out_ref.at[pl.ds(myq0 + STARTS[c], SIZES[c]), :],
                o_own.at[c],
            ).start()

        fx_list, fz_list = [], []
        for c in range(Kq):
            z_own[c].wait_recv()
            pltpu.make_async_copy(
                zrv.at[ds(c)],
                out_ref.at[pl.ds(qz * QR + STARTS[c], SIZES[c]), :],
                o_z.at[c],
            ).start()
            if c < KH:
                f = pltpu.make_async_remote_copy(
                    src_ref=zrv.at[ds(c)], dst_ref=dfx.at[ds(c)],
                    send_sem=fxsend.at[c], recv_sem=fxrecv.at[c],
                    device_id=px_id, device_id_type=pl.DeviceIdType.MESH,
                )
                f.start()
                fx_list.append(f)
            x_own[c].wait_recv()
            pltpu.make_async_copy(
                xrv.at[ds(c)],
                out_ref.at[pl.ds(qx * QR + STARTS[c], SIZES[c]), :],
                o_x.at[c],
            ).start()
            if c >= KH:
                j = c - KH
                f = pltpu.make_async_remote_copy(
                    src_ref=xrv.at[ds(c)], dst_ref=dfz.at[pl.ds(STARTS[c] - QR // 2, SIZES[c]), :],
                    send_sem=fzsend.at[j], recv_sem=fzrecv.at[j],
                    device_id=pz_id, device_id_type=pl.DeviceIdType.MESH,
                )
                f.start()
                fz_list.append(f)

        for j in range(Kq - KH):
            c2 = KH + j
            if j < KH:
                fx_list[j].wait_recv()
                pltpu.make_async_copy(
                    dfx.at[ds(j)],
                    out_ref.at[pl.ds(qd * QR + STARTS[j], SIZES[j]), :],
                    o_dx.at[j],
                ).start()
            fz_list[j].wait_recv()
            pltpu.make_async_copy(
                dfz.at[pl.ds(STARTS[c2] - QR // 2, SIZES[c2]), :],
                out_ref.at[pl.ds(qd * QR + STARTS[c2], SIZES[c2]), :],
                o_dz.at[j],
            ).start()

        for c in range(Kq):
            y_rdmas[c].wait_send()
            x_own[c].wait_send()
            z_own[c].wait_send()
            pltpu.make_async_copy(
                yv.at[ds(c)],
                out_ref.at[pl.ds(myq0 + STARTS[c], SIZES[c]), :],
                o_own.at[c],
            ).wait()
            pltpu.make_async_copy(
                zrv.at[ds(c)],
                out_ref.at[pl.ds(qz * QR + STARTS[c], SIZES[c]), :],
                o_z.at[c],
            ).wait()
            pltpu.make_async_copy(
                xrv.at[ds(c)],
                out_ref.at[pl.ds(qx * QR + STARTS[c], SIZES[c]), :],
                o_x.at[c],
            ).wait()
        for j in range(Kq - KH):
            c2 = KH + j
            if j < KH:
                fx_list[j].wait_send()
                pltpu.make_async_copy(
                    dfx.at[ds(j)],
                    out_ref.at[pl.ds(qd * QR + STARTS[j], SIZES[j]), :],
                    o_dx.at[j],
                ).wait()
            fz_list[j].wait_send()
            pltpu.make_async_copy(
                dfz.at[pl.ds(STARTS[c2] - QR // 2, SIZES[c2]), :],
                out_ref.at[pl.ds(qd * QR + STARTS[c2], SIZES[c2]), :],
                o_dz.at[j],
            ).wait()

    return pl.pallas_call(
        body,
        out_shape=jax.ShapeDtypeStruct((M, N), jnp.bfloat16),
        in_specs=[pl.BlockSpec(memory_space=pltpu.HBM)],
        out_specs=pl.BlockSpec(memory_space=pltpu.HBM),
        scratch_shapes=[
            pltpu.VMEM((QR, N), jnp.bfloat16),
            pltpu.VMEM((QR, N), jnp.bfloat16),
            pltpu.VMEM((QR, N), jnp.bfloat16),
            pltpu.VMEM((QR, N), jnp.bfloat16),
            pltpu.VMEM((QR // 2, N), jnp.bfloat16),
            pltpu.VMEM((QR // 2, N), jnp.bfloat16),
            pltpu.VMEM((2 * CHMAX, N), jnp.float32),
            pltpu.SemaphoreType.DMA((2,)),
            pltpu.SemaphoreType.DMA((Kq,)),
            pltpu.SemaphoreType.DMA((Kq,)),
            pltpu.SemaphoreType.DMA((Kq,)),
            pltpu.SemaphoreType.DMA((Kq,)),
            pltpu.SemaphoreType.DMA((Kq,)),
            pltpu.SemaphoreType.DMA((Kq,)),
            pltpu.SemaphoreType.DMA((KH,)),
            pltpu.SemaphoreType.DMA((KH,)),
            pltpu.SemaphoreType.DMA((Kq - KH,)),
            pltpu.SemaphoreType.DMA((Kq - KH,)),
            pltpu.SemaphoreType.DMA((Kq,)),
            pltpu.SemaphoreType.DMA((Kq,)),
            pltpu.SemaphoreType.DMA((Kq,)),
            pltpu.SemaphoreType.DMA((KH,)),
            pltpu.SemaphoreType.DMA((Kq - KH,)),
        ],
        compiler_params=pltpu.CompilerParams(
            collective_id=0,
            vmem_limit_bytes=62 * 1024 * 1024,
        ),
    )(x)
